# baseline (speedup 1.0000x reference)
import functools

import jax
import jax.numpy as jnp
from jax import lax
from jax.experimental import pallas as pl
from jax.experimental.pallas import tpu as pltpu
from jax.experimental.pallas import tpu_sc as plsc

N_NODES = 10000
N_EDGES = 320000
D_FEAT = 128
D_EDGE = 16

_NW = 32
_SPLITS = 2


_TC1_BLOCK = 1000


def _tc1_body(vd_ref, ws_ref, wr_ref, ps_ref, pr_ref):
    vd = vd_ref[...]
    ps_ref[...] = jnp.dot(vd, ws_ref[...], preferred_element_type=jnp.float32)
    pr_ref[...] = jnp.dot(vd, wr_ref[...], preferred_element_type=jnp.float32)


def _node_projections(vdata, ws_t, wr_t):
    grid = N_NODES // _TC1_BLOCK
    return pl.pallas_call(
        _tc1_body,
        grid=(grid,),
        in_specs=[
            pl.BlockSpec((_TC1_BLOCK, D_FEAT), lambda i: (i, 0)),
            pl.BlockSpec((D_FEAT, D_FEAT), lambda i: (0, 0)),
            pl.BlockSpec((D_FEAT, D_FEAT), lambda i: (0, 0)),
        ],
        out_specs=[
            pl.BlockSpec((_TC1_BLOCK, D_FEAT), lambda i: (i, 0)),
            pl.BlockSpec((_TC1_BLOCK, D_FEAT), lambda i: (i, 0)),
        ],
        out_shape=[
            jax.ShapeDtypeStruct((N_NODES, D_FEAT), jnp.float32),
            jax.ShapeDtypeStruct((N_NODES, D_FEAT), jnp.float32),
        ],
    )(vdata, ws_t, wr_t)


def _chunking(epw):
    for c in range(128, 0, -8):
        full = epw // c
        if full > 0 and full % 2 == 0 and epw - full * c <= c:
            return c, full, epw - full * c
    raise ValueError(epw)


def _sc_gather_sum(senders, receivers, ps, pr, ne):
    epw = ne // _NW
    c, full, tail = _chunking(epw)
    mesh = plsc.VectorSubcoreMesh(core_axis_name="c", subcore_axis_name="s")

    @functools.partial(
        pl.kernel,
        mesh=mesh,
        out_type=jax.ShapeDtypeStruct((ne, D_FEAT), jnp.float32),
        scratch_types=[
            pltpu.VMEM((epw,), jnp.int32),
            pltpu.VMEM((epw,), jnp.int32),
            pltpu.VMEM((c, D_FEAT), jnp.float32),
            pltpu.VMEM((c, D_FEAT), jnp.float32),
            pltpu.SemaphoreType.DMA,
            pltpu.SemaphoreType.DMA,
            pltpu.SemaphoreType.DMA,
            pltpu.SemaphoreType.DMA,
        ],
    )
    def k(sidx_hbm, ridx_hbm, ps_hbm, pr_hbm, out_hbm,
          sidx_v, ridx_v, rows0, rows1, semg0, semg1, semw0, semw1):
        wid = lax.axis_index("s") * 2 + lax.axis_index("c")
        base = wid * epw
        pltpu.sync_copy(sidx_hbm.at[pl.ds(base, epw)], sidx_v)
        pltpu.sync_copy(ridx_hbm.at[pl.ds(base, epw)], ridx_v)

        slots = ((rows0, semg0, semw0), (rows1, semg1, semw1))

        def gs(ci, rows, semg):
            pltpu.async_copy(ps_hbm.at[sidx_v.at[pl.ds(ci * c, c)]],
                             rows, semg)

        def ga(ci, rows, semg):
            pltpu.async_copy(pr_hbm.at[ridx_v.at[pl.ds(ci * c, c)]],
                             rows, semg, add=True)

        def wait_gather(rows, semg):
            pltpu.make_async_copy(ps_hbm.at[pl.ds(0, c)], rows, semg).wait()

        def wait_write(rows, semw):
            pltpu.make_async_copy(rows, out_hbm.at[pl.ds(0, c)], semw).wait()

        gs(0, rows0, semg0)
        gs(1, rows1, semg1)

        def pair_body(j, carry):
            for b, (rows, semg, semw) in enumerate(slots):
                ci = 2 * j + b
                wait_gather(rows, semg)
                ga(ci, rows, semg)
                wait_gather(rows, semg)
                pltpu.async_copy(rows, out_hbm.at[pl.ds(base + ci * c, c)],
                                 semw)

                @pl.when(ci + 2 < full)
                def _():
                    wait_write(rows, semw)
                    gs(ci + 2, rows, semg)

            return carry

        lax.fori_loop(0, full // 2, pair_body, 0)

        wait_write(rows0, semw0)
        wait_write(rows1, semw1)

        if tail:
            toff = full * c
            rows_t = rows0.at[pl.ds(0, tail)]
            pltpu.async_copy(
                ps_hbm.at[sidx_v.at[pl.ds(toff, tail)]], rows_t, semg0).wait()
            pltpu.async_copy(
                pr_hbm.at[ridx_v.at[pl.ds(toff, tail)]], rows_t, semg0,
                add=True).wait()
            pltpu.sync_copy(rows_t, out_hbm.at[pl.ds(base + toff, tail)])

    return k(senders, receivers, ps, pr)


_TC2_BLOCK = 4000


def _tc2_body(g_ref, ed_ref, we_ref, b_ref, out_ref):
    prod = jnp.dot(ed_ref[...], we_ref[...], preferred_element_type=jnp.float32)
    out_ref[...] = g_ref[...] + prod + b_ref[...]


def _edge_update(gathered, edata, we_t, b2d, ne):
    grid = ne // _TC2_BLOCK
    return pl.pallas_call(
        _tc2_body,
        grid=(grid,),
        in_specs=[
            pl.BlockSpec((_TC2_BLOCK, D_FEAT), lambda i: (i, 0)),
            pl.BlockSpec((_TC2_BLOCK, D_EDGE), lambda i: (i, 0)),
            pl.BlockSpec((D_EDGE, D_FEAT), lambda i: (0, 0)),
            pl.BlockSpec((1, D_FEAT), lambda i: (0, 0)),
        ],
        out_specs=pl.BlockSpec((_TC2_BLOCK, D_FEAT), lambda i: (i, 0)),
        out_shape=jax.ShapeDtypeStruct((ne, D_FEAT), jnp.float32),
    )(gathered, edata, we_t, b2d)


def kernel(vdata, edata, connectivity, W, b):
    senders = connectivity[0].astype(jnp.int32)
    receivers = connectivity[1].astype(jnp.int32)
    we_t = W[:, :D_EDGE].T
    ws_t = W[:, D_EDGE:D_EDGE + D_FEAT].T
    wr_t = W[:, D_EDGE + D_FEAT:].T
    b2d = b.reshape(1, D_FEAT)
    ps, pr = _node_projections(vdata, ws_t, wr_t)

    h = N_EDGES // _SPLITS
    outs = []
    for p in range(_SPLITS):
        sl = slice(p * h, (p + 1) * h)
        g = _sc_gather_sum(senders[sl], receivers[sl], ps, pr, h)
        outs.append(_edge_update(g, edata[sl], we_t, b2d, h))
    if _SPLITS == 1:
        return outs[0]
    return jnp.concatenate(outs, axis=0)


# speedup vs baseline: 1.2161x; 1.2161x over previous
import functools

import jax
import jax.numpy as jnp
from jax import lax
from jax.experimental import pallas as pl
from jax.experimental.pallas import tpu as pltpu
from jax.experimental.pallas import tpu_sc as plsc

N_NODES = 10000
N_EDGES = 320000
D_FEAT = 128
D_EDGE = 16

_NW = 32
_SPLITS = 1
_NSLOT = 4


_TC1_BLOCK = 1000


def _tc1_body(vd_ref, ws_ref, wr_ref, ps_ref, pr_ref):
    vd = vd_ref[...]
    ps_ref[...] = jnp.dot(vd, ws_ref[...], preferred_element_type=jnp.float32)
    pr_ref[...] = jnp.dot(vd, wr_ref[...], preferred_element_type=jnp.float32)


def _node_projections(vdata, ws_t, wr_t):
    grid = N_NODES // _TC1_BLOCK
    return pl.pallas_call(
        _tc1_body,
        grid=(grid,),
        in_specs=[
            pl.BlockSpec((_TC1_BLOCK, D_FEAT), lambda i: (i, 0)),
            pl.BlockSpec((D_FEAT, D_FEAT), lambda i: (0, 0)),
            pl.BlockSpec((D_FEAT, D_FEAT), lambda i: (0, 0)),
        ],
        out_specs=[
            pl.BlockSpec((_TC1_BLOCK, D_FEAT), lambda i: (i, 0)),
            pl.BlockSpec((_TC1_BLOCK, D_FEAT), lambda i: (i, 0)),
        ],
        out_shape=[
            jax.ShapeDtypeStruct((N_NODES, D_FEAT), jnp.float32),
            jax.ShapeDtypeStruct((N_NODES, D_FEAT), jnp.float32),
        ],
    )(vdata, ws_t, wr_t)


def _chunking(epw):
    for c in range(128, 0, -8):
        full = epw // c
        if full >= _NSLOT and epw - full * c <= c:
            return c, full, epw - full * c
    raise ValueError(epw)


def _sc_gather_sum(senders, receivers, ps, pr, ne):
    epw = ne // _NW
    c, full, tail = _chunking(epw)
    mesh = plsc.VectorSubcoreMesh(core_axis_name="c", subcore_axis_name="s")

    @functools.partial(
        pl.kernel,
        mesh=mesh,
        out_type=jax.ShapeDtypeStruct((ne, D_FEAT), jnp.float32),
        scratch_types=[
            pltpu.VMEM((epw,), jnp.int32),
            pltpu.VMEM((epw,), jnp.int32),
        ] + [pltpu.VMEM((c, D_FEAT), jnp.float32)] * _NSLOT
          + [pltpu.SemaphoreType.DMA] * (2 * _NSLOT),
    )
    def k(sidx_hbm, ridx_hbm, ps_hbm, pr_hbm, out_hbm,
          sidx_v, ridx_v, *bufs):
        rows_v = bufs[:_NSLOT]
        semg_v = bufs[_NSLOT:2 * _NSLOT]
        semw_v = bufs[2 * _NSLOT:]
        wid = lax.axis_index("s") * 2 + lax.axis_index("c")
        base = wid * epw
        pltpu.sync_copy(sidx_hbm.at[pl.ds(base, epw)], sidx_v)
        pltpu.sync_copy(ridx_hbm.at[pl.ds(base, epw)], ridx_v)

        def gs(ci, rows, semg):
            pltpu.async_copy(ps_hbm.at[sidx_v.at[pl.ds(ci * c, c)]],
                             rows, semg)

        def ga(ci, rows, semg):
            pltpu.async_copy(pr_hbm.at[ridx_v.at[pl.ds(ci * c, c)]],
                             rows, semg, add=True)

        def wait_gather(rows, semg):
            pltpu.make_async_copy(ps_hbm.at[pl.ds(0, c)], rows, semg).wait()

        def wait_write(rows, semw):
            pltpu.make_async_copy(rows, out_hbm.at[pl.ds(0, c)], semw).wait()

        for b in range(_NSLOT):
            gs(b, rows_v[b], semg_v[b])

        def ring_body(j, carry):
            for b in range(_NSLOT):
                rows, semg, semw = rows_v[b], semg_v[b], semw_v[b]
                ci = _NSLOT * j + b

                @pl.when(ci < full)
                def _():
                    wait_gather(rows, semg)
                    ga(ci, rows, semg)
                    wait_gather(rows, semg)
                    pltpu.async_copy(
                        rows, out_hbm.at[pl.ds(base + ci * c, c)], semw)

                    @pl.when(ci + _NSLOT < full)
                    def _():
                        wait_write(rows, semw)
                        gs(ci + _NSLOT, rows, semg)

            return carry

        lax.fori_loop(0, (full + _NSLOT - 1) // _NSLOT, ring_body, 0)

        for b in range(_NSLOT):
            wait_write(rows_v[b], semw_v[b])

        if tail:
            toff = full * c
            rows_t = rows_v[0].at[pl.ds(0, tail)]
            pltpu.async_copy(
                ps_hbm.at[sidx_v.at[pl.ds(toff, tail)]], rows_t,
                semg_v[0]).wait()
            pltpu.async_copy(
                pr_hbm.at[ridx_v.at[pl.ds(toff, tail)]], rows_t,
                semg_v[0], add=True).wait()
            pltpu.sync_copy(rows_t, out_hbm.at[pl.ds(base + toff, tail)])

    return k(senders, receivers, ps, pr)


_TC2_BLOCK = 4000


def _tc2_body(g_ref, ed_ref, we_ref, b_ref, out_ref):
    prod = jnp.dot(ed_ref[...], we_ref[...], preferred_element_type=jnp.float32)
    out_ref[...] = g_ref[...] + prod + b_ref[...]


def _edge_update(gathered, edata, we_t, b2d, ne):
    grid = ne // _TC2_BLOCK
    return pl.pallas_call(
        _tc2_body,
        grid=(grid,),
        in_specs=[
            pl.BlockSpec((_TC2_BLOCK, D_FEAT), lambda i: (i, 0)),
            pl.BlockSpec((_TC2_BLOCK, D_EDGE), lambda i: (i, 0)),
            pl.BlockSpec((D_EDGE, D_FEAT), lambda i: (0, 0)),
            pl.BlockSpec((1, D_FEAT), lambda i: (0, 0)),
        ],
        out_specs=pl.BlockSpec((_TC2_BLOCK, D_FEAT), lambda i: (i, 0)),
        out_shape=jax.ShapeDtypeStruct((ne, D_FEAT), jnp.float32),
    )(gathered, edata, we_t, b2d)


def kernel(vdata, edata, connectivity, W, b):
    senders = connectivity[0].astype(jnp.int32)
    receivers = connectivity[1].astype(jnp.int32)
    we_t = W[:, :D_EDGE].T
    ws_t = W[:, D_EDGE:D_EDGE + D_FEAT].T
    wr_t = W[:, D_EDGE + D_FEAT:].T
    b2d = b.reshape(1, D_FEAT)
    ps, pr = _node_projections(vdata, ws_t, wr_t)

    h = N_EDGES // _SPLITS
    outs = []
    for p in range(_SPLITS):
        sl = slice(p * h, (p + 1) * h)
        g = _sc_gather_sum(senders[sl], receivers[sl], ps, pr, h)
        outs.append(_edge_update(g, edata[sl], we_t, b2d, h))
    if _SPLITS == 1:
        return outs[0]
    return jnp.concatenate(outs, axis=0)
